# baseline (speedup 1.0000x reference)
import functools

import jax
import jax.numpy as jnp
from jax import lax
from jax.experimental import pallas as pl
from jax.experimental.pallas import tpu as pltpu
from jax.experimental.pallas import tpu_sc as plsc

V = 100000
N_NEW = 200
D = 128

NC = 2
NS = 16
NW = NC * NS

CHUNK = 128
LANES = 16


def _make_gather(total_rows: int):
    per_w = total_rows // NW
    n_chunks = per_w // CHUNK
    assert n_chunks % 2 == 0 and n_chunks >= 4
    mesh = plsc.VectorSubcoreMesh(core_axis_name="c", subcore_axis_name="s")

    @functools.partial(
        pl.kernel,
        mesh=mesh,
        out_type=jax.ShapeDtypeStruct((total_rows, D), jnp.float32),
        scratch_types=[
            pltpu.VMEM((per_w,), jnp.int32),
            pltpu.VMEM((CHUNK,), jnp.int32),
            pltpu.VMEM((CHUNK,), jnp.int32),
            pltpu.VMEM((CHUNK, D), jnp.float32),
            pltpu.VMEM((CHUNK, D), jnp.float32),
            pltpu.VMEM((N_NEW, D), jnp.float32),
            pltpu.VMEM((LANES,), jnp.int32),
            pltpu.SemaphoreType.DMA,
            pltpu.SemaphoreType.DMA,
            pltpu.SemaphoreType.DMA,
            pltpu.SemaphoreType.DMA,
            pltpu.SemaphoreType.DMA,
        ],
    )
    def gather_kernel(idx_hbm, orig_hbm, new_hbm, out_hbm,
                      allidx, mn0, mn1, rows0, rows1, newtab, accv,
                      in0, in1, out0, out1, sem_fix):
        wid = lax.axis_index("s") * NC + lax.axis_index("c")
        base = wid * per_w
        mn = (mn0, mn1)
        rows = (rows0, rows1)
        sem_in = (in0, in1)
        sem_out = (out0, out1)

        pltpu.sync_copy(idx_hbm.at[pl.ds(base, per_w)], allidx)
        pltpu.sync_copy(new_hbm, newtab)
        accv[...] = jnp.zeros((LANES,), jnp.int32)

        def fire_gather(g, b):
            acc = accv[...]
            for j in range(CHUNK // LANES):
                sl = raw_slice(g, j)
                mn[b][pl.ds(j * LANES, LANES)] = jnp.minimum(sl, V - 1)
                acc = acc | jnp.where(sl >= V, 1, 0)
            accv[...] = acc
            pltpu.async_copy(orig_hbm.at[mn[b]], rows[b], sem_in[b])

        def raw_slice(g, j):
            return allidx[pl.ds(g * CHUNK + j * LANES, LANES)]

        def fire_out(g, b):
            off = base + g * CHUNK
            pltpu.async_copy(rows[b], out_hbm.at[pl.ds(off, CHUNK)],
                             sem_out[b])

        def wait_gather(b):
            pltpu.make_async_copy(orig_hbm.at[mn[b]], rows[b],
                                  sem_in[b]).wait()

        def wait_out(g, b):
            off = base + g * CHUNK
            pltpu.make_async_copy(rows[b], out_hbm.at[pl.ds(off, CHUNK)],
                                  sem_out[b]).wait()

        def any_scalar(ov_i32):
            m = ov_i32
            iota = lax.iota(jnp.int32, LANES)
            for sh in (8, 4, 2, 1):
                perm = iota ^ sh
                m = m | m.at[perm].get(mode="promise_in_bounds")
            return m[0] != 0

        def ov_vec(idx_vec):
            return jnp.where(idx_vec >= V, 1, 0)

        fire_gather(0, 0)
        fire_gather(1, 1)
        wait_gather(0)
        fire_out(0, 0)

        def body(outer, carry):
            for b in range(2):
                g = 2 * outer + b
                wait_out(g - 2, b)
                fire_gather(g, b)
                wait_gather(1 - b)
                fire_out(g - 1, 1 - b)
            return carry

        lax.fori_loop(1, n_chunks // 2, body, 0, unroll=False)

        wait_gather(1)
        fire_out(n_chunks - 1, 1)
        wait_out(n_chunks - 2, 0)
        wait_out(n_chunks - 1, 1)

        @pl.when(any_scalar(accv[...]))
        def _():
            def chunk_body(g, carry):
                acc = ov_vec(raw_slice(g, 0))
                for j in range(1, CHUNK // LANES):
                    acc = acc | ov_vec(raw_slice(g, j))

                @pl.when(any_scalar(acc))
                def _():
                    def group_body(t, carry2):
                        vt = allidx[pl.ds(g * CHUNK + t * LANES, LANES)]
                        ovt = ov_vec(vt)
                        vn = vt - V
                        for i in range(LANES):
                            @pl.when(ovt[i] != 0)
                            def _(i=i):
                                idxn = vn[i]
                                row = base + g * CHUNK + t * LANES + i
                                pltpu.async_copy(
                                    newtab.at[pl.ds(idxn, 1)],
                                    out_hbm.at[pl.ds(row, 1)],
                                    sem_fix)
                                pltpu.make_async_copy(
                                    newtab.at[pl.ds(idxn, 1)],
                                    out_hbm.at[pl.ds(row, 1)],
                                    sem_fix).wait()
                        return carry2

                    lax.fori_loop(0, CHUNK // LANES, group_body, 0,
                                  unroll=False)
                return carry

            lax.fori_loop(0, n_chunks, chunk_body, 0, unroll=False)

    return gather_kernel


def kernel(x, original_weight, new_embedding):
    idx = x.reshape(-1).astype(jnp.int32)
    out = _make_gather(idx.shape[0])(idx, original_weight, new_embedding)
    return out.reshape(x.shape + (D,))


# speedup vs baseline: 1.0549x; 1.0549x over previous
import functools

import jax
import jax.numpy as jnp
from jax import lax
from jax.experimental import pallas as pl
from jax.experimental.pallas import tpu as pltpu
from jax.experimental.pallas import tpu_sc as plsc

V = 100000
N_NEW = 200
D = 128

NC = 2
NS = 16
NW = NC * NS

CHUNK = 128
LANES = 16


def _make_gather(total_rows: int):
    per_w = total_rows // NW
    n_chunks = per_w // CHUNK
    assert n_chunks % 2 == 0 and n_chunks >= 4
    mesh = plsc.VectorSubcoreMesh(core_axis_name="c", subcore_axis_name="s")

    @functools.partial(
        pl.kernel,
        mesh=mesh,
        out_type=jax.ShapeDtypeStruct((total_rows, D), jnp.float32),
        scratch_types=[
            pltpu.VMEM((per_w,), jnp.int32),
            pltpu.VMEM((CHUNK,), jnp.int32),
            pltpu.VMEM((CHUNK,), jnp.int32),
            pltpu.VMEM((CHUNK, D), jnp.float32),
            pltpu.VMEM((CHUNK, D), jnp.float32),
            pltpu.VMEM((N_NEW, D), jnp.float32),
            pltpu.VMEM((LANES,), jnp.int32),
            pltpu.VMEM((n_chunks, LANES), jnp.int32),
            pltpu.SemaphoreType.DMA,
            pltpu.SemaphoreType.DMA,
            pltpu.SemaphoreType.DMA,
            pltpu.SemaphoreType.DMA,
            pltpu.SemaphoreType.DMA,
        ],
    )
    def gather_kernel(idx_hbm, orig_hbm, new_hbm, out_hbm,
                      allidx, mn0, mn1, rows0, rows1, newtab, accv, flags,
                      in0, in1, out0, out1, sem_fix):
        wid = lax.axis_index("s") * NC + lax.axis_index("c")
        base = wid * per_w
        mn = (mn0, mn1)
        rows = (rows0, rows1)
        sem_in = (in0, in1)
        sem_out = (out0, out1)

        pltpu.sync_copy(idx_hbm.at[pl.ds(base, per_w)], allidx)
        pltpu.sync_copy(new_hbm, newtab)
        accv[...] = jnp.zeros((LANES,), jnp.int32)

        def fire_gather(g, b):
            acc = jnp.zeros((LANES,), jnp.int32)
            for j in range(CHUNK // LANES):
                sl = raw_slice(g, j)
                mn[b][pl.ds(j * LANES, LANES)] = jnp.minimum(sl, V - 1)
                acc = acc | jnp.where(sl >= V, 1, 0)
            flags[g, pl.ds(0, LANES)] = acc
            accv[...] = accv[...] | acc
            pltpu.async_copy(orig_hbm.at[mn[b]], rows[b], sem_in[b])

        def raw_slice(g, j):
            return allidx[pl.ds(g * CHUNK + j * LANES, LANES)]

        def fire_out(g, b):
            off = base + g * CHUNK
            pltpu.async_copy(rows[b], out_hbm.at[pl.ds(off, CHUNK)],
                             sem_out[b])

        def wait_gather(b):
            pltpu.make_async_copy(orig_hbm.at[mn[b]], rows[b],
                                  sem_in[b]).wait()

        def wait_out(g, b):
            off = base + g * CHUNK
            pltpu.make_async_copy(rows[b], out_hbm.at[pl.ds(off, CHUNK)],
                                  sem_out[b]).wait()

        def any_scalar(ov_i32):
            m = ov_i32
            iota = lax.iota(jnp.int32, LANES)
            for sh in (8, 4, 2, 1):
                perm = iota ^ sh
                m = m | m.at[perm].get(mode="promise_in_bounds")
            return m[0] != 0

        def ov_vec(idx_vec):
            return jnp.where(idx_vec >= V, 1, 0)

        fire_gather(0, 0)
        fire_gather(1, 1)
        wait_gather(0)
        fire_out(0, 0)

        def body(outer, carry):
            for b in range(2):
                g = 2 * outer + b
                wait_out(g - 2, b)
                fire_gather(g, b)
                wait_gather(1 - b)
                fire_out(g - 1, 1 - b)
            return carry

        lax.fori_loop(1, n_chunks // 2, body, 0, unroll=False)

        wait_gather(1)
        fire_out(n_chunks - 1, 1)
        wait_out(n_chunks - 2, 0)
        wait_out(n_chunks - 1, 1)

        @pl.when(any_scalar(accv[...]))
        def _():
            def chunk_body(g, cnt):
                def checked(cnt):
                    def group_body(t, cnt2):
                        vt = allidx[pl.ds(g * CHUNK + t * LANES, LANES)]
                        ovt = ov_vec(vt)

                        def lanes_scan(cnt3):
                            vn = vt - V
                            c = cnt3
                            for i in range(LANES):
                                c = c + ovt[i]

                                @pl.when(ovt[i] != 0)
                                def _(i=i):
                                    idxn = vn[i]
                                    row = (base + g * CHUNK
                                           + t * LANES + i)
                                    pltpu.async_copy(
                                        newtab.at[pl.ds(idxn, 1)],
                                        out_hbm.at[pl.ds(row, 1)],
                                        sem_fix)
                            return c

                        return lax.cond(any_scalar(ovt), lanes_scan,
                                        lambda c: c, cnt2)

                    return lax.fori_loop(0, CHUNK // LANES, group_body,
                                         cnt, unroll=False)

                return lax.cond(any_scalar(flags[g, pl.ds(0, LANES)]),
                                checked, lambda c: c, cnt)

            n_fix = lax.fori_loop(0, n_chunks, chunk_body, 0,
                                  unroll=False)

            def drain(_, carry):
                pltpu.make_async_copy(newtab.at[pl.ds(0, 1)],
                                      out_hbm.at[pl.ds(base, 1)],
                                      sem_fix).wait()
                return carry

            lax.fori_loop(0, n_fix, drain, 0, unroll=False)

    return gather_kernel


def kernel(x, original_weight, new_embedding):
    idx = x.reshape(-1).astype(jnp.int32)
    out = _make_gather(idx.shape[0])(idx, original_weight, new_embedding)
    return out.reshape(x.shape + (D,))


# speedup vs baseline: 1.0595x; 1.0044x over previous
import functools

import jax
import jax.numpy as jnp
from jax import lax
from jax.experimental import pallas as pl
from jax.experimental.pallas import tpu as pltpu
from jax.experimental.pallas import tpu_sc as plsc

V = 100000
N_NEW = 200
D = 128

NC = 2
NS = 16
NW = NC * NS

CHUNK = 128
LANES = 16


def _make_gather(total_rows: int):
    per_w = total_rows // NW
    n_chunks = per_w // CHUNK
    assert n_chunks % 2 == 0 and n_chunks >= 4
    mesh = plsc.VectorSubcoreMesh(core_axis_name="c", subcore_axis_name="s")

    @functools.partial(
        pl.kernel,
        mesh=mesh,
        out_type=jax.ShapeDtypeStruct((total_rows, D), jnp.float32),
        scratch_types=[
            pltpu.VMEM((per_w,), jnp.int32),
            pltpu.VMEM((CHUNK,), jnp.int32),
            pltpu.VMEM((CHUNK,), jnp.int32),
            pltpu.VMEM((CHUNK, D), jnp.float32),
            pltpu.VMEM((CHUNK, D), jnp.float32),
            pltpu.VMEM((N_NEW, D), jnp.float32),
            pltpu.VMEM((LANES,), jnp.int32),
            pltpu.VMEM((n_chunks, LANES), jnp.int32),
            pltpu.SemaphoreType.DMA,
            pltpu.SemaphoreType.DMA,
            pltpu.SemaphoreType.DMA,
            pltpu.SemaphoreType.DMA,
            pltpu.SemaphoreType.DMA,
        ],
    )
    def gather_kernel(idx_hbm, orig_hbm, new_hbm, out_hbm,
                      allidx, mn0, mn1, rows0, rows1, newtab, accv, flags,
                      in0, in1, out0, out1, sem_fix):
        wid = lax.axis_index("s") * NC + lax.axis_index("c")
        base = wid * per_w
        mn = (mn0, mn1)
        rows = (rows0, rows1)
        sem_in = (in0, in1)
        sem_out = (out0, out1)

        pltpu.sync_copy(idx_hbm.at[pl.ds(base, per_w)], allidx)
        pltpu.sync_copy(new_hbm, newtab)
        accv[...] = jnp.zeros((LANES,), jnp.int32)

        def fire_gather(g, b):
            acc = jnp.zeros((LANES,), jnp.int32)
            for j in range(CHUNK // LANES):
                sl = raw_slice(g, j)
                mn[b][pl.ds(j * LANES, LANES)] = jnp.minimum(sl, V - 1)
                acc = acc | jnp.where(sl >= V, 1 << j, 0)
            flags[g, pl.ds(0, LANES)] = acc
            accv[...] = accv[...] | acc
            pltpu.async_copy(orig_hbm.at[mn[b]], rows[b], sem_in[b])

        def raw_slice(g, j):
            return allidx[pl.ds(g * CHUNK + j * LANES, LANES)]

        def fire_out(g, b):
            off = base + g * CHUNK
            pltpu.async_copy(rows[b], out_hbm.at[pl.ds(off, CHUNK)],
                             sem_out[b])

        def wait_gather(b):
            pltpu.make_async_copy(orig_hbm.at[mn[b]], rows[b],
                                  sem_in[b]).wait()

        def wait_out(g, b):
            off = base + g * CHUNK
            pltpu.make_async_copy(rows[b], out_hbm.at[pl.ds(off, CHUNK)],
                                  sem_out[b]).wait()

        def or_all_lanes(v_i32):
            m = v_i32
            iota = lax.iota(jnp.int32, LANES)
            for sh in (8, 4, 2, 1):
                perm = iota ^ sh
                m = m | m.at[perm].get(mode="promise_in_bounds")
            return m[0]

        def any_scalar(ov_i32):
            return or_all_lanes(ov_i32) != 0

        def ov_vec(idx_vec):
            return jnp.where(idx_vec >= V, 1, 0)

        fire_gather(0, 0)
        fire_gather(1, 1)
        wait_gather(0)
        fire_out(0, 0)

        def body(outer, carry):
            for b in range(2):
                g = 2 * outer + b
                wait_out(g - 2, b)
                fire_gather(g, b)
                wait_gather(1 - b)
                fire_out(g - 1, 1 - b)
            return carry

        lax.fori_loop(1, n_chunks // 2, body, 0, unroll=False)

        wait_gather(1)
        fire_out(n_chunks - 1, 1)
        wait_out(n_chunks - 2, 0)
        wait_out(n_chunks - 1, 1)

        @pl.when(any_scalar(accv[...]))
        def _():
            def chunk_body(g, cnt):
                smask = or_all_lanes(flags[g, pl.ds(0, LANES)])

                def checked(cnt):
                    c = cnt
                    for j in range(CHUNK // LANES):
                        def lanes_scan(cnt3, j=j):
                            vt = allidx[
                                pl.ds(g * CHUNK + j * LANES, LANES)]
                            ovt = ov_vec(vt)
                            vn = vt - V
                            c3 = cnt3
                            for i in range(LANES):
                                c3 = c3 + ovt[i]

                                @pl.when(ovt[i] != 0)
                                def _(i=i, j=j):
                                    idxn = vn[i]
                                    row = (base + g * CHUNK
                                           + j * LANES + i)
                                    pltpu.async_copy(
                                        newtab.at[pl.ds(idxn, 1)],
                                        out_hbm.at[pl.ds(row, 1)],
                                        sem_fix)
                            return c3

                        c = lax.cond((smask >> j) & 1 != 0, lanes_scan,
                                     lambda c3: c3, c)
                    return c

                return lax.cond(smask != 0, checked, lambda c: c, cnt)

            n_fix = lax.fori_loop(0, n_chunks, chunk_body, 0,
                                  unroll=False)

            def drain(_, carry):
                pltpu.make_async_copy(newtab.at[pl.ds(0, 1)],
                                      out_hbm.at[pl.ds(base, 1)],
                                      sem_fix).wait()
                return carry

            lax.fori_loop(0, n_fix, drain, 0, unroll=False)

    return gather_kernel


def kernel(x, original_weight, new_embedding):
    idx = x.reshape(-1).astype(jnp.int32)
    out = _make_gather(idx.shape[0])(idx, original_weight, new_embedding)
    return out.reshape(x.shape + (D,))


# speedup vs baseline: 1.0641x; 1.0044x over previous
import functools

import jax
import jax.numpy as jnp
from jax import lax
from jax.experimental import pallas as pl
from jax.experimental.pallas import tpu as pltpu
from jax.experimental.pallas import tpu_sc as plsc

V = 100000
N_NEW = 200
D = 128

NC = 2
NS = 16
NW = NC * NS

CHUNK = 128
LANES = 16


def _make_gather(total_rows: int):
    per_w = total_rows // NW
    n_chunks = per_w // CHUNK
    assert n_chunks % 2 == 0 and n_chunks >= 4
    mesh = plsc.VectorSubcoreMesh(core_axis_name="c", subcore_axis_name="s")

    @functools.partial(
        pl.kernel,
        mesh=mesh,
        out_type=jax.ShapeDtypeStruct((total_rows, D), jnp.float32),
        scratch_types=[
            pltpu.VMEM((per_w,), jnp.int32),
            pltpu.VMEM((CHUNK,), jnp.int32),
            pltpu.VMEM((CHUNK,), jnp.int32),
            pltpu.VMEM((CHUNK,), jnp.int32),
            pltpu.VMEM((CHUNK,), jnp.int32),
            pltpu.VMEM((CHUNK, D), jnp.float32),
            pltpu.VMEM((CHUNK, D), jnp.float32),
            pltpu.VMEM((CHUNK, D), jnp.float32),
            pltpu.VMEM((CHUNK, D), jnp.float32),
            pltpu.VMEM((N_NEW, D), jnp.float32),
            pltpu.VMEM((LANES,), jnp.int32),
            pltpu.VMEM((n_chunks, LANES), jnp.int32),
            pltpu.SemaphoreType.DMA,
            pltpu.SemaphoreType.DMA,
            pltpu.SemaphoreType.DMA,
            pltpu.SemaphoreType.DMA,
            pltpu.SemaphoreType.DMA,
            pltpu.SemaphoreType.DMA,
            pltpu.SemaphoreType.DMA,
            pltpu.SemaphoreType.DMA,
            pltpu.SemaphoreType.DMA,
        ],
    )
    def gather_kernel(idx_hbm, orig_hbm, new_hbm, out_hbm,
                      allidx, mn0, mn1, mn2, mn3,
                      rows0, rows1, rows2, rows3, newtab, accv, flags,
                      in0, in1, in2, in3, out0, out1, out2, out3,
                      sem_fix):
        wid = lax.axis_index("s") * NC + lax.axis_index("c")
        base = wid * per_w
        mn = (mn0, mn1, mn2, mn3)
        rows = (rows0, rows1, rows2, rows3)
        sem_in = (in0, in1, in2, in3)
        sem_out = (out0, out1, out2, out3)

        pltpu.sync_copy(idx_hbm.at[pl.ds(base, per_w)], allidx)
        pltpu.sync_copy(new_hbm, newtab)
        accv[...] = jnp.zeros((LANES,), jnp.int32)

        def fire_gather(g, b):
            acc = jnp.zeros((LANES,), jnp.int32)
            for j in range(CHUNK // LANES):
                sl = raw_slice(g, j)
                mn[b][pl.ds(j * LANES, LANES)] = jnp.minimum(sl, V - 1)
                acc = acc | jnp.where(sl >= V, 1 << j, 0)
            flags[g, pl.ds(0, LANES)] = acc
            accv[...] = accv[...] | acc
            pltpu.async_copy(orig_hbm.at[mn[b]], rows[b], sem_in[b])

        def raw_slice(g, j):
            return allidx[pl.ds(g * CHUNK + j * LANES, LANES)]

        def fire_out(g, b):
            off = base + g * CHUNK
            pltpu.async_copy(rows[b], out_hbm.at[pl.ds(off, CHUNK)],
                             sem_out[b])

        def wait_gather(b):
            pltpu.make_async_copy(orig_hbm.at[mn[b]], rows[b],
                                  sem_in[b]).wait()

        def wait_out(g, b):
            off = base + g * CHUNK
            pltpu.make_async_copy(rows[b], out_hbm.at[pl.ds(off, CHUNK)],
                                  sem_out[b]).wait()

        def or_all_lanes(v_i32):
            m = v_i32
            iota = lax.iota(jnp.int32, LANES)
            for sh in (8, 4, 2, 1):
                perm = iota ^ sh
                m = m | m.at[perm].get(mode="promise_in_bounds")
            return m[0]

        def any_scalar(ov_i32):
            return or_all_lanes(ov_i32) != 0

        def ov_vec(idx_vec):
            return jnp.where(idx_vec >= V, 1, 0)

        assert (n_chunks - 2) % 4 == 0
        for g in range(4):
            fire_gather(g, g)
        for g in range(3):
            wait_gather(g)
            fire_out(g, g)

        def body(k, carry):
            for b in range(4):
                g = 4 * k + b
                wait_out(g - 4, b)
                fire_gather(g, b)
                wait_gather((b - 1) % 4)
                fire_out(g - 1, (b - 1) % 4)
            return carry

        lax.fori_loop(1, (n_chunks - 2) // 4, body, 0, unroll=False)

        n = n_chunks
        wait_out(n - 6, 0)
        fire_gather(n - 2, 0)
        wait_gather(3)
        fire_out(n - 3, 3)
        wait_out(n - 5, 1)
        fire_gather(n - 1, 1)
        wait_gather(0)
        fire_out(n - 2, 0)
        wait_gather(1)
        fire_out(n - 1, 1)
        wait_out(n - 4, 2)
        wait_out(n - 3, 3)
        wait_out(n - 2, 0)
        wait_out(n - 1, 1)

        @pl.when(any_scalar(accv[...]))
        def _():
            def chunk_body(g, cnt):
                smask = or_all_lanes(flags[g, pl.ds(0, LANES)])

                def checked(cnt):
                    c = cnt
                    for j in range(CHUNK // LANES):
                        def lanes_scan(cnt3, j=j):
                            vt = allidx[
                                pl.ds(g * CHUNK + j * LANES, LANES)]
                            ovt = ov_vec(vt)
                            vn = vt - V
                            c3 = cnt3
                            for i in range(LANES):
                                c3 = c3 + ovt[i]

                                @pl.when(ovt[i] != 0)
                                def _(i=i, j=j):
                                    idxn = vn[i]
                                    row = (base + g * CHUNK
                                           + j * LANES + i)
                                    pltpu.async_copy(
                                        newtab.at[pl.ds(idxn, 1)],
                                        out_hbm.at[pl.ds(row, 1)],
                                        sem_fix)
                            return c3

                        c = lax.cond((smask >> j) & 1 != 0, lanes_scan,
                                     lambda c3: c3, c)
                    return c

                return lax.cond(smask != 0, checked, lambda c: c, cnt)

            n_fix = lax.fori_loop(0, n_chunks, chunk_body, 0,
                                  unroll=False)

            def drain(_, carry):
                pltpu.make_async_copy(newtab.at[pl.ds(0, 1)],
                                      out_hbm.at[pl.ds(base, 1)],
                                      sem_fix).wait()
                return carry

            lax.fori_loop(0, n_fix, drain, 0, unroll=False)

    return gather_kernel


def kernel(x, original_weight, new_embedding):
    idx = x.reshape(-1).astype(jnp.int32)
    out = _make_gather(idx.shape[0])(idx, original_weight, new_embedding)
    return out.reshape(x.shape + (D,))
